# baseline (speedup 1.0000x reference)
import functools

import jax
import jax.numpy as jnp
from jax import lax
from jax.experimental import pallas as pl
from jax.experimental.pallas import tpu as pltpu
from jax.experimental.pallas import tpu_sc as plsc

_NC = 128
_NS = 2048
_K = 4
_LANES = 16
_CSTAGE = 4


def _rsqrt_nr(r2):
    i = lax.bitcast_convert_type(r2, jnp.int32)
    i = jnp.int32(0x5F3759DF) - lax.shift_right_logical(i, 1)
    y = lax.bitcast_convert_type(i, jnp.float32)
    hr = r2 * jnp.float32(0.5)
    for _ in range(3):
        y = y * (jnp.float32(1.5) - hr * y * y)
    return y


def _make_sc_kernel(n_pix):
    info = plsc.get_sparse_core_info()
    nw = info.num_cores * info.num_subcores
    assert n_pix % (nw * _LANES) == 0
    px_w = n_pix // nw
    n_pv = px_w // _LANES
    n_stage = _NC // _CSTAGE
    mesh = plsc.VectorSubcoreMesh(core_axis_name="c", subcore_axis_name="s")

    @functools.partial(
        pl.kernel,
        out_type=jax.ShapeDtypeStruct((nw, _K, px_w), jnp.float32),
        mesh=mesh,
        compiler_params=pltpu.CompilerParams(needs_layout_passes=False),
        scratch_types=[
            pltpu.VMEM((_CSTAGE * _K * _NS,), jnp.float32),
            pltpu.VMEM((_CSTAGE * _K * _NS,), jnp.float32),
            pltpu.VMEM((_K, px_w), jnp.float32),
            pltpu.VMEM((px_w,), jnp.float32),
            pltpu.VMEM((px_w,), jnp.float32),
            pltpu.VMEM((_NC, _LANES), jnp.float32),
            pltpu.VMEM((2, _LANES), jnp.float32),
            pltpu.SemaphoreType.DMA,
            pltpu.SemaphoreType.DMA,
        ],
    )
    def k(rf_h, x_h, z_h, xr_h, p_h, out_h,
          rf_buf0, rf_buf1, acc, xb, zb, xrb, pb, sem0, sem1):
        rf_bufs = (rf_buf0, rf_buf1)
        stage_len = _CSTAGE * _K * _NS
        wid = lax.axis_index("s") * info.num_cores + lax.axis_index("c")
        base = wid * px_w
        pltpu.sync_copy(x_h.at[pl.ds(base, px_w)], xb)
        pltpu.sync_copy(z_h.at[pl.ds(base, px_w)], zb)
        pltpu.sync_copy(xr_h, xrb)
        pltpu.sync_copy(p_h, pb)

        def zero_body(j, c):
            for kk in range(_K):
                acc[kk, pl.ds(j * _LANES, _LANES)] = jnp.zeros(
                    (_LANES,), jnp.float32)
            return c
        lax.fori_loop(0, n_pv, zero_body, 0)

        alpha_v = pb[0]
        beta_v = pb[1]

        pltpu.async_copy(rf_h.at[pl.ds(0, stage_len)], rf_buf0, sem0)

        def compute_stage(s, par):
            rf_v = rf_bufs[par]

            @plsc.parallel_loop(0, n_pv, step=1, unroll=4)
            def pv_body(pv):
                o = pv * _LANES
                xv = xb[pl.ds(o, _LANES)]
                zv = zb[pl.ds(o, _LANES)]
                zz = zv * zv
                zterm = alpha_v * zv + beta_v
                accs = [acc[kk, pl.ds(o, _LANES)] for kk in range(_K)]
                for cc in range(_CSTAGE):
                    xr = xrb[s * _CSTAGE + cc]
                    dx = xv - xr
                    r2 = jnp.maximum(dx * dx + zz, jnp.float32(1e-30))
                    rr = r2 * _rsqrt_nr(r2)
                    sv = rr * alpha_v + zterm
                    xc = jnp.minimum(
                        jnp.maximum(sv, jnp.float32(0.0)),
                        jnp.float32(_NS - 1))
                    i0 = jnp.minimum(xc.astype(jnp.int32),
                                     jnp.int32(_NS - 2))
                    tf = xc - i0.astype(jnp.float32)
                    ib = lax.shift_left(i0, 2) + jnp.int32(cc * _NS * _K)
                    for kk in range(_K):
                        j0 = ib + jnp.int32(kk)
                        y0 = plsc.load_gather(rf_v, [j0])
                        y1 = plsc.load_gather(rf_v, [j0 + _K])
                        accs[kk] = accs[kk] + (y0 + tf * (y1 - y0))
                for kk in range(_K):
                    acc[kk, pl.ds(o, _LANES)] = accs[kk]

        def super_body(u, c):
            s0 = 2 * u
            pltpu.make_async_copy(
                rf_h.at[pl.ds(s0 * stage_len, stage_len)], rf_buf0,
                sem0).wait()
            pltpu.async_copy(
                rf_h.at[pl.ds((s0 + 1) * stage_len, stage_len)], rf_buf1,
                sem1)
            compute_stage(s0, 0)
            pltpu.make_async_copy(
                rf_h.at[pl.ds((s0 + 1) * stage_len, stage_len)], rf_buf1,
                sem1).wait()

            @pl.when(s0 + 2 < n_stage)
            def _():
                pltpu.async_copy(
                    rf_h.at[pl.ds((s0 + 2) * stage_len, stage_len)],
                    rf_buf0, sem0)
            compute_stage(s0 + 1, 1)
            return c
        lax.fori_loop(0, n_stage // 2, super_body, 0)

        pltpu.sync_copy(acc, out_h.at[wid])

    return k, nw, px_w


def kernel(rf, g, pr, p):
    b, nc, ns, kf = rf.shape
    nz, nx = g.shape[1], g.shape[2]
    n_pix = nz * nx
    sc_k, nw, px_w = _make_sc_kernel(n_pix)
    outs = []
    for bi in range(b):
        rf_t = rf[bi].reshape(-1)
        xf = g[bi, :, :, 0].reshape(-1)
        zf = g[bi, :, :, 2].reshape(-1)
        xr_b = jnp.broadcast_to(pr[bi, :, 0][:, None], (nc, _LANES))
        c0, fs, t0 = p[bi, 0], p[bi, 1], p[bi, 2]
        alpha = fs / c0
        beta = fs * t0 / c0
        pb = jnp.stack([jnp.full((_LANES,), 1.0, jnp.float32) * alpha,
                        jnp.full((_LANES,), 1.0, jnp.float32) * beta])
        out = sc_k(rf_t, xf, zf, xr_b, pb)
        img = out.transpose(0, 2, 1).reshape(nz, nx, kf)
        outs.append(img)
    return jnp.stack(outs)


# speedup vs baseline: 2.1066x; 2.1066x over previous
import functools

import jax
import jax.numpy as jnp
from jax import lax
from jax.experimental import pallas as pl
from jax.experimental.pallas import tpu as pltpu
from jax.experimental.pallas import tpu_sc as plsc

_NC = 128
_NS = 2048
_K = 4
_LANES = 16
_CSTAGE = 4


def _rsqrt_nr(r2):
    i = lax.bitcast_convert_type(r2, jnp.int32)
    i = jnp.int32(0x5F1FFFF9) - lax.shift_right_logical(i, 1)
    y = lax.bitcast_convert_type(i, jnp.float32)
    y = y * (jnp.float32(1.68191391) - jnp.float32(0.703952009) * r2 * y * y)
    y = y * (jnp.float32(1.5) - jnp.float32(0.5) * r2 * y * y)
    return y


def _make_sc_kernel(n_pix):
    info = plsc.get_sparse_core_info()
    nw = info.num_cores * info.num_subcores
    assert n_pix % (nw * _LANES) == 0
    px_w = n_pix // nw
    n_pv = px_w // _LANES
    n_stage = _NC // _CSTAGE
    mesh = plsc.VectorSubcoreMesh(core_axis_name="c", subcore_axis_name="s")

    @functools.partial(
        pl.kernel,
        out_type=jax.ShapeDtypeStruct((nw, _K, px_w), jnp.float32),
        mesh=mesh,
        compiler_params=pltpu.CompilerParams(needs_layout_passes=False),
        scratch_types=[
            pltpu.VMEM((_CSTAGE * _K * _NS,), jnp.float32),
            pltpu.VMEM((_CSTAGE * _K * _NS,), jnp.float32),
            pltpu.VMEM((_K, px_w), jnp.float32),
            pltpu.VMEM((px_w,), jnp.float32),
            pltpu.VMEM((px_w,), jnp.float32),
            pltpu.VMEM((_NC, _LANES), jnp.float32),
            pltpu.VMEM((2, _LANES), jnp.float32),
            pltpu.SemaphoreType.DMA,
            pltpu.SemaphoreType.DMA,
        ],
    )
    def k(rf_h, x_h, z_h, xr_h, p_h, out_h,
          rf_buf0, rf_buf1, acc, xb, zb, xrb, pb, sem0, sem1):
        rf_bufs = (rf_buf0, rf_buf1)
        stage_len = _CSTAGE * _K * _NS
        wid = lax.axis_index("s") * info.num_cores + lax.axis_index("c")
        base = wid * px_w
        pltpu.sync_copy(x_h.at[pl.ds(base, px_w)], xb)
        pltpu.sync_copy(z_h.at[pl.ds(base, px_w)], zb)
        pltpu.sync_copy(xr_h, xrb)
        pltpu.sync_copy(p_h, pb)

        def zero_body(j, c):
            for kk in range(_K):
                acc[kk, pl.ds(j * _LANES, _LANES)] = jnp.zeros(
                    (_LANES,), jnp.float32)
            return c
        lax.fori_loop(0, n_pv, zero_body, 0)

        alpha_v = pb[0]
        beta_v = pb[1]

        pltpu.async_copy(rf_h.at[pl.ds(0, stage_len)], rf_buf0, sem0)

        def compute_stage(s, par):
            rf_v = rf_bufs[par]
            xrs = [xrb[s * _CSTAGE + cc] for cc in range(_CSTAGE)]

            @plsc.parallel_loop(0, n_pv, step=1, unroll=2)
            def pv_body(pv):
                o = pv * _LANES
                xv = xb[pl.ds(o, _LANES)]
                zv = zb[pl.ds(o, _LANES)]
                zz = zv * zv
                zterm = alpha_v * zv + beta_v
                accs = [acc[kk, pl.ds(o, _LANES)] for kk in range(_K)]
                for cc in range(_CSTAGE):
                    xr = xrb[s * _CSTAGE + cc]
                    dx = xv - xr
                    r2 = jnp.maximum(dx * dx + zz, jnp.float32(1e-30))
                    rr = r2 * _rsqrt_nr(r2)
                    sv = rr * alpha_v + zterm
                    xc = jnp.minimum(
                        jnp.maximum(sv, jnp.float32(0.0)),
                        jnp.float32(_NS - 1))
                    i0 = jnp.minimum(xc.astype(jnp.int32),
                                     jnp.int32(_NS - 2))
                    tf = xc - i0.astype(jnp.float32)
                    i1 = i0 + 1
                    for kk in range(_K):
                        ref = rf_v.at[pl.ds((cc * _K + kk) * _NS, _NS)]
                        y0 = plsc.load_gather(ref, [i0])
                        y1 = plsc.load_gather(ref, [i1])
                        accs[kk] = accs[kk] + (y0 + tf * (y1 - y0))
                for kk in range(_K):
                    acc[kk, pl.ds(o, _LANES)] = accs[kk]

        def super_body(u, c):
            s0 = 2 * u
            pltpu.make_async_copy(
                rf_h.at[pl.ds(s0 * stage_len, stage_len)], rf_buf0,
                sem0).wait()
            pltpu.async_copy(
                rf_h.at[pl.ds((s0 + 1) * stage_len, stage_len)], rf_buf1,
                sem1)
            compute_stage(s0, 0)
            pltpu.make_async_copy(
                rf_h.at[pl.ds((s0 + 1) * stage_len, stage_len)], rf_buf1,
                sem1).wait()

            @pl.when(s0 + 2 < n_stage)
            def _():
                pltpu.async_copy(
                    rf_h.at[pl.ds((s0 + 2) * stage_len, stage_len)],
                    rf_buf0, sem0)
            compute_stage(s0 + 1, 1)
            return c
        lax.fori_loop(0, n_stage // 2, super_body, 0)

        pltpu.sync_copy(acc, out_h.at[wid])

    return k, nw, px_w


def kernel(rf, g, pr, p):
    b, nc, ns, kf = rf.shape
    nz, nx = g.shape[1], g.shape[2]
    n_pix = nz * nx
    sc_k, nw, px_w = _make_sc_kernel(n_pix)
    outs = []
    for bi in range(b):
        rf_t = jnp.transpose(rf[bi], (0, 2, 1)).reshape(-1)
        xf = g[bi, :, :, 0].reshape(-1)
        zf = g[bi, :, :, 2].reshape(-1)
        xr_b = jnp.broadcast_to(pr[bi, :, 0][:, None], (nc, _LANES))
        c0, fs, t0 = p[bi, 0], p[bi, 1], p[bi, 2]
        alpha = fs / c0
        beta = fs * t0 / c0
        pb = jnp.stack([jnp.full((_LANES,), 1.0, jnp.float32) * alpha,
                        jnp.full((_LANES,), 1.0, jnp.float32) * beta])
        out = sc_k(rf_t, xf, zf, xr_b, pb)
        img = out.transpose(0, 2, 1).reshape(nz, nx, kf)
        outs.append(img)
    return jnp.stack(outs)
